# baseline (device time: 8844 ns/iter reference)
import jax
import jax.numpy as jnp
from jax import lax
from jax.experimental import pallas as pl
from jax.experimental.pallas import tpu as pltpu

Z = 4


def kernel(x, pi):
    def body(x_ref, pi_ref, out_ref, send_buf, send_sem, recv_sem):
        my_x = lax.axis_index("x")
        my_y = lax.axis_index("y")
        my_z = lax.axis_index("z")

        tgt = pi_ref[my_z]
        src = jnp.int32(0)
        for j in range(Z):
            src = jnp.where(pi_ref[j] == my_z, jnp.int32(j), src)

        barrier = pltpu.get_barrier_semaphore()
        pl.semaphore_signal(
            barrier,
            inc=1,
            device_id=(my_x, my_y, src),
            device_id_type=pl.DeviceIdType.MESH,
        )

        half = x_ref.shape[1] // 2
        send_buf[:, :half, :] = x_ref[:, :half, :].astype(jnp.bfloat16)
        pl.semaphore_wait(barrier, 1)

        rdma0 = pltpu.make_async_remote_copy(
            src_ref=send_buf.at[:, :half, :],
            dst_ref=out_ref.at[:, :half, :],
            send_sem=send_sem.at[0],
            recv_sem=recv_sem.at[0],
            device_id=(my_x, my_y, tgt),
            device_id_type=pl.DeviceIdType.MESH,
        )
        rdma0.start()

        send_buf[:, half:, :] = x_ref[:, half:, :].astype(jnp.bfloat16)
        rdma1 = pltpu.make_async_remote_copy(
            src_ref=send_buf.at[:, half:, :],
            dst_ref=out_ref.at[:, half:, :],
            send_sem=send_sem.at[1],
            recv_sem=recv_sem.at[1],
            device_id=(my_x, my_y, tgt),
            device_id_type=pl.DeviceIdType.MESH,
        )
        rdma1.start()

        rdma0.wait()
        rdma1.wait()

    return pl.pallas_call(
        body,
        out_shape=jax.ShapeDtypeStruct(x.shape, jnp.bfloat16),
        in_specs=[
            pl.BlockSpec(memory_space=pltpu.VMEM),
            pl.BlockSpec(memory_space=pltpu.SMEM),
        ],
        out_specs=pl.BlockSpec(memory_space=pltpu.VMEM),
        scratch_shapes=[
            pltpu.VMEM(x.shape, jnp.bfloat16),
            pltpu.SemaphoreType.DMA((2,)),
            pltpu.SemaphoreType.DMA((2,)),
        ],
        compiler_params=pltpu.CompilerParams(collective_id=0),
    )(x, pi)
